# baseline (speedup 1.0000x reference)
import functools

import jax
import jax.numpy as jnp
from jax import lax
from jax.experimental import pallas as pl
from jax.experimental.pallas import tpu as pltpu
from jax.experimental.pallas import tpu_sc as plsc

N_NODES = 10000
D = 128
E = 320000

NC = 2
NS = 16

CHUNK = 128
N_CHUNKS = E // CHUNK
T0 = 1433
T1 = N_CHUNKS - T0
N_ACC = 10112
ROWS_PER_TILE = N_ACC // NS


def _sc_aggregate(x, ei):
    mesh = plsc.VectorSubcoreMesh(core_axis_name="c", subcore_axis_name="s")

    @functools.partial(
        pl.kernel,
        mesh=mesh,
        out_type=jax.ShapeDtypeStruct((NC, N_ACC, D), jnp.float32),
        scratch_types=[
            pltpu.VMEM((CHUNK, D), jnp.float32),
            pltpu.VMEM((CHUNK, D), jnp.float32),
            pltpu.VMEM((CHUNK, D), jnp.float32),
            pltpu.VMEM((2, CHUNK), jnp.int32),
            pltpu.VMEM((2, CHUNK), jnp.int32),
            pltpu.VMEM((2, CHUNK), jnp.int32),
            pltpu.VMEM_SHARED((N_ACC, D), jnp.float32),
            pltpu.SemaphoreType.DMA,
            pltpu.SemaphoreType.DMA,
            pltpu.SemaphoreType.DMA,
            pltpu.SemaphoreType.DMA,
            pltpu.SemaphoreType.DMA,
            pltpu.SemaphoreType.DMA,
        ],
    )
    def body(x_hbm, ei_hbm, z_hbm, out_hbm,
             buf0, buf1, buf2, ib0, ib1, ib2, acc_sh,
             sem0, sem1, sem2, isem0, isem1, isem2):
        cid = lax.axis_index("c")
        sid = lax.axis_index("s")
        bufs = (buf0, buf1, buf2)
        sems = (sem0, sem1, sem2)
        ibs = (ib0, ib1, ib2)
        isems = (isem0, isem1, isem2)

        pltpu.sync_copy(z_hbm, buf0)
        r0 = sid * ROWS_PER_TILE
        for b in range(ROWS_PER_TILE // CHUNK):
            pltpu.sync_copy(buf0, acc_sh.at[pl.ds(r0 + b * CHUNK, CHUNK)])
        rem = ROWS_PER_TILE % CHUNK
        if rem:
            nfull = ROWS_PER_TILE // CHUNK
            pltpu.sync_copy(buf0.at[pl.ds(0, rem)],
                            acc_sh.at[pl.ds(r0 + nfull * CHUNK, rem)])
        plsc.subcore_barrier()

        q0, rm0 = T0 // NS, T0 % NS
        q1, rm1 = T1 // NS, T1 % NS
        s32 = sid.astype(jnp.int32)
        start0 = s32 * q0 + jnp.minimum(s32, rm0)
        cnt0 = q0 + jnp.where(s32 < rm0, 1, 0)
        start1 = T0 + s32 * q1 + jnp.minimum(s32, rm1)
        cnt1 = q1 + jnp.where(s32 < rm1, 1, 0)
        start = jnp.where(cid == 0, start0, start1)
        cnt = jnp.where(cid == 0, cnt0, cnt1)

        def load_idx(copy, chunk_i, k):
            copy(ei_hbm.at[pl.ds(0, 2), pl.ds(chunk_i * CHUNK, CHUNK)],
                 ibs[k])

        def wait_idx(k):
            pltpu.make_async_copy(ei_hbm.at[pl.ds(0, 2), pl.ds(0, CHUNK)],
                                  ibs[k], isems[k]).wait()

        load_idx(pltpu.sync_copy, start, 0)
        load_idx(lambda s, d: pltpu.async_copy(s, d, isem1), start + 1, 1)
        load_idx(lambda s, d: pltpu.async_copy(s, d, isem2), start + 2, 2)
        pltpu.async_copy(x_hbm.at[ib0.at[0]], buf0, sem0)
        wait_idx(1)
        pltpu.async_copy(x_hbm.at[ib1.at[0]], buf1, sem1)

        def triple(t, carry):
            for p in range(3):
                i = 3 * t + p
                pn = (p + 2) % 3

                @pl.when(i < cnt)
                def _consume(p=p, i=i):
                    pltpu.make_async_copy(x_hbm.at[pl.ds(0, CHUNK)],
                                          bufs[p], sems[p]).wait()
                    pltpu.sync_copy(bufs[p], acc_sh.at[ibs[p].at[1]],
                                    add=True)

                @pl.when(i + 3 < cnt)
                def _load(p=p, i=i):
                    load_idx(lambda s, d, p=p: pltpu.async_copy(s, d,
                                                                isems[p]),
                             start + i + 3, p)

                @pl.when(i + 2 < cnt)
                def _gather(pn=pn, i=i):
                    wait_idx(pn)
                    pltpu.async_copy(x_hbm.at[ibs[pn].at[0]], bufs[pn],
                                     sems[pn])
            return carry

        lax.fori_loop(0, (cnt + 2) // 3, triple, 0)
        plsc.subcore_barrier()
        plsc.subcore_barrier()

        pltpu.sync_copy(acc_sh.at[pl.ds(r0, ROWS_PER_TILE)],
                        out_hbm.at[cid, pl.ds(r0, ROWS_PER_TILE)])

    zblock = jnp.zeros((CHUNK, D), jnp.float32)
    return body(x, ei, zblock)


def _dense_r_body(x_ref, wr_ref, b_ref, o_ref):
    o_ref[...] = lax.dot_general(
        x_ref[...], wr_ref[...], (((1,), (1,)), ((), ())),
        preferred_element_type=jnp.float32) + b_ref[...]


def _dense_l_body(p0_ref, p1_ref, yr_ref, wl_ref, o_ref):
    agg = p0_ref[0] + p1_ref[0]
    o_ref[...] = lax.dot_general(
        agg, wl_ref[...], (((1,), (1,)), ((), ())),
        preferred_element_type=jnp.float32) + yr_ref[...]


def kernel(x, edge_index, W_l, b_l, W_r):
    blk = 1000
    grid = (N_NODES // blk,)

    y_r = pl.pallas_call(
        _dense_r_body,
        grid=grid,
        in_specs=[
            pl.BlockSpec((blk, D), lambda i: (i, 0)),
            pl.BlockSpec((D, D), lambda i: (0, 0)),
            pl.BlockSpec((1, D), lambda i: (0, 0)),
        ],
        out_specs=pl.BlockSpec((blk, D), lambda i: (i, 0)),
        out_shape=jax.ShapeDtypeStruct((N_NODES, D), jnp.float32),
    )(x, W_r, b_l.reshape(1, D))

    p = _sc_aggregate(x, edge_index)

    out = pl.pallas_call(
        _dense_l_body,
        grid=grid,
        in_specs=[
            pl.BlockSpec((1, blk, D), lambda i: (0, i, 0)),
            pl.BlockSpec((1, blk, D), lambda i: (1, i, 0)),
            pl.BlockSpec((blk, D), lambda i: (i, 0)),
            pl.BlockSpec((D, D), lambda i: (0, 0)),
        ],
        out_specs=pl.BlockSpec((blk, D), lambda i: (i, 0)),
        out_shape=jax.ShapeDtypeStruct((N_NODES, D), jnp.float32),
    )(p, p, y_r, W_l)
    return out


# speedup vs baseline: 1.0814x; 1.0814x over previous
import functools

import jax
import jax.numpy as jnp
from jax import lax
from jax.experimental import pallas as pl
from jax.experimental.pallas import tpu as pltpu
from jax.experimental.pallas import tpu_sc as plsc

N_NODES = 10000
D = 128
E = 320000

NC = 2
NS = 16

CHUNK = 128
N_CHUNKS = E // CHUNK
T0 = 1270
T1 = N_CHUNKS - T0
N_ACC = 10112
ROWS_PER_TILE = N_ACC // NS


def _sc_aggregate(x, ei):
    mesh = plsc.VectorSubcoreMesh(core_axis_name="c", subcore_axis_name="s")

    @functools.partial(
        pl.kernel,
        mesh=mesh,
        out_type=jax.ShapeDtypeStruct((NC, N_ACC, D), jnp.float32),
        scratch_types=[
            pltpu.VMEM((CHUNK, D), jnp.float32),
            pltpu.VMEM((CHUNK, D), jnp.float32),
            pltpu.VMEM((CHUNK, D), jnp.float32),
            pltpu.VMEM((2, CHUNK), jnp.int32),
            pltpu.VMEM((2, CHUNK), jnp.int32),
            pltpu.VMEM((2, CHUNK), jnp.int32),
            pltpu.VMEM_SHARED((N_ACC, D), jnp.float32),
            pltpu.SemaphoreType.DMA,
            pltpu.SemaphoreType.DMA,
            pltpu.SemaphoreType.DMA,
            pltpu.SemaphoreType.DMA,
            pltpu.SemaphoreType.DMA,
            pltpu.SemaphoreType.DMA,
        ],
    )
    def body(x_hbm, ei_hbm, z_hbm, out_hbm,
             buf0, buf1, buf2, ib0, ib1, ib2, acc_sh,
             sem0, sem1, sem2, isem0, isem1, isem2):
        cid = lax.axis_index("c")
        sid = lax.axis_index("s")
        bufs = (buf0, buf1, buf2)
        sems = (sem0, sem1, sem2)
        ibs = (ib0, ib1, ib2)
        isems = (isem0, isem1, isem2)

        pltpu.sync_copy(z_hbm, buf0)
        r0 = sid * ROWS_PER_TILE
        for b in range(ROWS_PER_TILE // CHUNK):
            pltpu.sync_copy(buf0, acc_sh.at[pl.ds(r0 + b * CHUNK, CHUNK)])
        rem = ROWS_PER_TILE % CHUNK
        if rem:
            nfull = ROWS_PER_TILE // CHUNK
            pltpu.sync_copy(buf0.at[pl.ds(0, rem)],
                            acc_sh.at[pl.ds(r0 + nfull * CHUNK, rem)])
        plsc.subcore_barrier()

        q0, rm0 = T0 // NS, T0 % NS
        q1, rm1 = T1 // NS, T1 % NS
        s32 = sid.astype(jnp.int32)
        start0 = s32 * q0 + jnp.minimum(s32, rm0)
        cnt0 = q0 + jnp.where(s32 < rm0, 1, 0)
        start1 = T0 + s32 * q1 + jnp.minimum(s32, rm1)
        cnt1 = q1 + jnp.where(s32 < rm1, 1, 0)
        start = jnp.where(cid == 0, start0, start1)
        cnt = jnp.where(cid == 0, cnt0, cnt1)

        def load_idx(copy, chunk_i, k):
            copy(ei_hbm.at[pl.ds(0, 2), pl.ds(chunk_i * CHUNK, CHUNK)],
                 ibs[k])

        def wait_idx(k):
            pltpu.make_async_copy(ei_hbm.at[pl.ds(0, 2), pl.ds(0, CHUNK)],
                                  ibs[k], isems[k]).wait()

        load_idx(pltpu.sync_copy, start, 0)
        load_idx(lambda s, d: pltpu.async_copy(s, d, isem1), start + 1, 1)
        load_idx(lambda s, d: pltpu.async_copy(s, d, isem2), start + 2, 2)
        pltpu.async_copy(x_hbm.at[ib0.at[0]], buf0, sem0)
        wait_idx(1)
        pltpu.async_copy(x_hbm.at[ib1.at[0]], buf1, sem1)

        def triple(t, carry):
            for p in range(3):
                i = 3 * t + p
                pn = (p + 2) % 3

                @pl.when(i < cnt)
                def _consume(p=p, i=i):
                    pltpu.make_async_copy(x_hbm.at[pl.ds(0, CHUNK)],
                                          bufs[p], sems[p]).wait()
                    pltpu.sync_copy(bufs[p], acc_sh.at[ibs[p].at[1]],
                                    add=True)

                @pl.when(i + 3 < cnt)
                def _load(p=p, i=i):
                    load_idx(lambda s, d, p=p: pltpu.async_copy(s, d,
                                                                isems[p]),
                             start + i + 3, p)

                @pl.when(i + 2 < cnt)
                def _gather(pn=pn, i=i):
                    wait_idx(pn)
                    pltpu.async_copy(x_hbm.at[ibs[pn].at[0]], bufs[pn],
                                     sems[pn])
            return carry

        lax.fori_loop(0, (cnt + 2) // 3, triple, 0)
        plsc.subcore_barrier()
        plsc.subcore_barrier()

        pltpu.sync_copy(acc_sh.at[pl.ds(r0, ROWS_PER_TILE)],
                        out_hbm.at[cid, pl.ds(r0, ROWS_PER_TILE)])

    zblock = jnp.zeros((CHUNK, D), jnp.float32)
    return body(x, ei, zblock)


def _dense_r_body(x_ref, wr_ref, b_ref, o_ref):
    o_ref[...] = lax.dot_general(
        x_ref[...], wr_ref[...], (((1,), (1,)), ((), ())),
        preferred_element_type=jnp.float32) + b_ref[...]


def _dense_l_body(p0_ref, p1_ref, yr_ref, wl_ref, o_ref):
    agg = p0_ref[0] + p1_ref[0]
    o_ref[...] = lax.dot_general(
        agg, wl_ref[...], (((1,), (1,)), ((), ())),
        preferred_element_type=jnp.float32) + yr_ref[...]


def kernel(x, edge_index, W_l, b_l, W_r):
    blk = 1000
    grid = (N_NODES // blk,)

    y_r = pl.pallas_call(
        _dense_r_body,
        grid=grid,
        in_specs=[
            pl.BlockSpec((blk, D), lambda i: (i, 0)),
            pl.BlockSpec((D, D), lambda i: (0, 0)),
            pl.BlockSpec((1, D), lambda i: (0, 0)),
        ],
        out_specs=pl.BlockSpec((blk, D), lambda i: (i, 0)),
        out_shape=jax.ShapeDtypeStruct((N_NODES, D), jnp.float32),
    )(x, W_r, b_l.reshape(1, D))

    p = _sc_aggregate(x, edge_index)

    out = pl.pallas_call(
        _dense_l_body,
        grid=grid,
        in_specs=[
            pl.BlockSpec((1, blk, D), lambda i: (0, i, 0)),
            pl.BlockSpec((1, blk, D), lambda i: (1, i, 0)),
            pl.BlockSpec((blk, D), lambda i: (i, 0)),
            pl.BlockSpec((D, D), lambda i: (0, 0)),
        ],
        out_specs=pl.BlockSpec((blk, D), lambda i: (i, 0)),
        out_shape=jax.ShapeDtypeStruct((N_NODES, D), jnp.float32),
    )(p, p, y_r, W_l)
    return out
